# baseline (speedup 1.0000x reference)
import functools

import jax
import jax.numpy as jnp
from jax import lax
from jax.experimental import pallas as pl
from jax.experimental.pallas import tpu as pltpu, tpu_sc as plsc

SEQ = 200
EXPAND = 16
K_CH = 8
N_CH = SEQ // K_CH
ROWS_PER_CHUNK = 16
IDX_MINOR = 128
IDX_ROWS = ROWS_PER_CHUNK * SEQ // IDX_MINOR
REP_ROWS = ROWS_PER_CHUNK * SEQ


def _rank_body(ts_ref, out_ref):
    ts = ts_ref[...]
    b = ts.shape[0]
    sb = lax.bitcast_convert_type(ts, jnp.int32)
    s = jnp.where(sb < 0, sb ^ jnp.int32(0x7FFFFFFF), sb)

    s3 = s[:, None, :]
    jiota = lax.broadcasted_iota(jnp.int32, (1, K_CH, SEQ), 2)
    kiota = lax.broadcasted_iota(jnp.int32, (1, K_CH, SEQ), 1)
    acc = jnp.zeros((b, K_CH, SEQ), jnp.int32)
    for q in range(N_CH):
        sk = lax.slice(s, (0, q * K_CH), (b, (q + 1) * K_CH))
        m = (jiota > kiota + (q * K_CH)).astype(jnp.int32)
        cmp = (sk[:, :, None] - m) < s3
        acc = acc + cmp.astype(jnp.int32)
    rank = jnp.sum(acc, axis=1)

    base = pl.program_id(0) * b
    row = lax.broadcasted_iota(jnp.int32, (b, 1), 0) + base
    out_ref[...] = rank + row * SEQ


@functools.partial(jax.jit, static_argnames=("block_b",))
def _ranks(ts, block_b=64):
    batch = ts.shape[0]
    grid = (batch // block_b,)
    return pl.pallas_call(
        _rank_body,
        grid=grid,
        in_specs=[pl.BlockSpec((block_b, SEQ), lambda i: (i, 0))],
        out_specs=pl.BlockSpec((block_b, SEQ), lambda i: (i, 0)),
        out_shape=jax.ShapeDtypeStruct((batch, SEQ), jnp.int32),
    )(ts)


def _make_scatter(n_chunks):
    info = plsc.get_sparse_core_info()
    nw = info.num_cores * info.num_subcores
    chunks_per_w = n_chunks // nw
    out_rows = n_chunks * REP_ROWS
    mesh = plsc.VectorSubcoreMesh(core_axis_name="c", subcore_axis_name="s")

    @functools.partial(
        pl.kernel,
        mesh=mesh,
        compiler_params=pltpu.CompilerParams(use_tc_tiling_on_sc=False),
        out_type=jax.ShapeDtypeStruct((out_rows, EXPAND), jnp.float32),
        scratch_types=[
            pltpu.VMEM((REP_ROWS, EXPAND), jnp.float32),
            pltpu.VMEM((IDX_ROWS, IDX_MINOR), jnp.int32),
            pltpu.VMEM((IDX_ROWS, IDX_MINOR), jnp.int32),
            pltpu.SemaphoreType.DMA,
            pltpu.SemaphoreType.DMA,
        ],
    )
    def scatter(idx_hbm, tab_hbm, out_hbm, rep_v, idx0_v, idx1_v, sem0, sem1):
        wid = lax.axis_index("s") * info.num_cores + lax.axis_index("c")
        c0 = wid * chunks_per_w
        for p in range(ROWS_PER_CHUNK):
            pltpu.sync_copy(tab_hbm, rep_v.at[pl.ds(p * SEQ, SEQ)])
        pltpu.sync_copy(idx_hbm.at[c0], idx0_v)
        pltpu.sync_copy(idx_hbm.at[c0 + 1], idx1_v)

        def fire(idx_v, sem):
            for j in range(IDX_ROWS):
                pltpu.make_async_copy(
                    rep_v.at[pl.ds(j * IDX_MINOR, IDX_MINOR)],
                    out_hbm.at[idx_v.at[j]],
                    sem,
                ).start()

        def drain(sem):
            pltpu.make_async_copy(
                out_hbm.at[pl.ds(0, REP_ROWS)], rep_v, sem
            ).wait()

        def body(g, carry):
            ca = c0 + 2 * g
            fire(idx0_v, sem0)
            fire(idx1_v, sem1)
            drain(sem0)
            na = jnp.minimum(ca + 2, c0 + chunks_per_w - 1)
            pltpu.sync_copy(idx_hbm.at[na], idx0_v)
            drain(sem1)
            nb = jnp.minimum(ca + 3, c0 + chunks_per_w - 1)
            pltpu.sync_copy(idx_hbm.at[nb], idx1_v)
            return carry

        lax.fori_loop(0, chunks_per_w // 2, body, 0)

    return scatter


@jax.jit
def _scatter_out(idx3, table):
    n_chunks = idx3.shape[0]
    return _make_scatter(n_chunks)(idx3, table)


def kernel(ts, pos_embeddings):
    batch = ts.shape[0]
    idx = _ranks(ts)
    idx3 = idx.reshape(batch // ROWS_PER_CHUNK, IDX_ROWS, IDX_MINOR)
    out = _scatter_out(idx3, pos_embeddings)
    return out.reshape(batch, SEQ, EXPAND)


# speedup vs baseline: 5.5981x; 5.5981x over previous
import functools

import jax
import jax.numpy as jnp
from jax import lax
from jax.experimental import pallas as pl

SEQ = 200
EXPAND = 16
K_CH = 8
IG = 8


def _body(tsT_ref, tabT_ref, out_ref):
    sT = tsT_ref[...]
    bsz = sT.shape[1]
    sb = lax.bitcast_convert_type(sT, jnp.int32)
    s = jnp.where(sb < 0, sb ^ jnp.int32(0x7FFFFFFF), sb)

    jio3 = lax.broadcasted_iota(jnp.int32, (K_CH, SEQ, 1), 1)
    kio3 = lax.broadcasted_iota(jnp.int32, (K_CH, SEQ, 1), 0)
    acc = jnp.zeros((SEQ, bsz), jnp.int32)
    for q in range(SEQ // K_CH):
        sk = lax.slice(s, (q * K_CH, 0), ((q + 1) * K_CH, bsz))
        m3 = (jio3 > kio3 + (q * K_CH)).astype(jnp.int32)
        cmp = (sk[:, None, :] - m3) < s[None]
        acc = acc + jnp.sum(cmp.astype(jnp.int32), axis=0)

    tabT = tabT_ref[...]
    rt = jnp.concatenate([acc] * IG, axis=1)
    lane = lax.broadcasted_iota(jnp.int32, (1, IG * bsz), 1)
    for ig in range(SEQ // IG):
        ii = lane // bsz + (ig * IG)
        mi = (rt == ii).astype(jnp.float32)
        og = lax.dot_general(
            tabT, mi, (((1,), (0,)), ((), ())),
            preferred_element_type=jnp.float32,
        )
        for t in range(IG):
            out_ref[ig * IG + t, :, :] = lax.slice(
                og, (0, t * bsz), (EXPAND, (t + 1) * bsz)
            )


@functools.partial(jax.jit, static_argnames=("block_b",))
def _run(tsT, tabT, block_b=512):
    batch = tsT.shape[1]
    grid = (batch // block_b,)
    return pl.pallas_call(
        _body,
        grid=grid,
        in_specs=[
            pl.BlockSpec((SEQ, block_b), lambda i: (0, i)),
            pl.BlockSpec((EXPAND, SEQ), lambda i: (0, 0)),
        ],
        out_specs=pl.BlockSpec((SEQ, EXPAND, block_b), lambda i: (0, 0, i)),
        out_shape=jax.ShapeDtypeStruct((SEQ, EXPAND, batch), jnp.float32),
    )(tsT, tabT)


def kernel(ts, pos_embeddings):
    outT = _run(ts.T, pos_embeddings.T)
    return jnp.transpose(outT, (2, 0, 1))
